# Initial kernel scaffold; baseline (speedup 1.0000x reference)
import jax
import jax.numpy as jnp
from jax.experimental import pallas as pl


def kernel(marginals, lens):
    raise NotImplementedError("write your pallas kernel here")


# speedup vs baseline: 120.3168x; 120.3168x over previous
import functools

import jax
import jax.numpy as jnp
from jax import lax
from jax.experimental import pallas as pl
from jax.experimental.pallas import tpu as pltpu
from jax.experimental.pallas import tpu_sc as plsc

B, N = 32, 128
L = 16
NEG = -1e9
NUM_CORES = 2
NUM_SUBCORES = 16


def _cky_body(marg_hbm, lens_hbm, out_hbm, marg_v, sw_v, lens_v, root_v):
    wid = lax.axis_index("s") * NUM_CORES + lax.axis_index("c")

    pltpu.sync_copy(marg_hbm.at[wid], marg_v)
    pltpu.sync_copy(lens_hbm, lens_v)

    iota = lax.iota(jnp.int32, L)
    nmax = jnp.int32(N - 1)

    negv = jnp.full((L,), NEG, jnp.float32)
    for c in range(N // L):
        sw_v[pl.ds(L * c, L)] = negv

    for c in range(N // L):
        iv = iota + L * c
        jv = jnp.minimum(iv + 1, nmax)
        sw_v[pl.ds(N + L * c, L)] = plsc.load_gather(marg_v, [iv * N + jv])

    def make_band(nb):
        def band_body(w, carry):
            def a_body(a, accs):
                by = a * N
                bz = (w - a) * N + a
                out = []
                for c in range(nb):
                    y = sw_v[pl.ds(by + L * c, L)]
                    z = sw_v[pl.ds(bz + L * c, L)]
                    out.append(jnp.maximum(accs[c], y + z))
                return tuple(out)

            accs0 = tuple(jnp.full((L,), NEG, jnp.float32) for _ in range(nb))
            accs = lax.fori_loop(1, w, a_body, accs0)
            for c in range(nb):
                iv = iota + L * c
                jv = jnp.minimum(iv + w, nmax)
                d = plsc.load_gather(marg_v, [iv * N + jv])
                sw_v[pl.ds(w * N + L * c, L)] = accs[c] + d
            return carry

        return band_body

    for nb in range(N // L, 0, -1):
        w_lo = max(2, N - L * nb)
        w_hi = N - L * (nb - 1) - 1
        lax.fori_loop(w_lo, w_hi + 1, make_band(nb), jnp.int32(0))

    widx = jnp.full((L,), wid, jnp.int32)
    lv = plsc.load_gather(lens_v, [widx])
    root_v[...] = plsc.load_gather(sw_v, [lv * N])
    pltpu.sync_copy(root_v, out_hbm.at[wid])


@jax.jit
def _cky_sc(marginals, lens):
    mesh = plsc.VectorSubcoreMesh(
        core_axis_name="c", subcore_axis_name="s",
        num_cores=NUM_CORES, num_subcores=NUM_SUBCORES,
    )
    k = functools.partial(
        pl.kernel,
        out_type=jax.ShapeDtypeStruct((B, L), jnp.float32),
        mesh=mesh,
        compiler_params=pltpu.CompilerParams(needs_layout_passes=False),
        scratch_types=[
            pltpu.VMEM((N * N,), jnp.float32),
            pltpu.VMEM((N * N,), jnp.float32),
            pltpu.VMEM((B,), jnp.int32),
            pltpu.VMEM((L,), jnp.float32),
        ],
    )(_cky_body)
    return k(marginals.reshape(B, N * N), lens.astype(jnp.int32))


def kernel(marginals, lens):
    out = _cky_sc(marginals, lens)
    return out[:, 0]


# speedup vs baseline: 123.3969x; 1.0256x over previous
import functools

import jax
import jax.numpy as jnp
from jax import lax
from jax.experimental import pallas as pl
from jax.experimental.pallas import tpu as pltpu
from jax.experimental.pallas import tpu_sc as plsc

B, N = 32, 128
L = 16
U = 8
G = U - 2
NEG = -1e9
NUM_CORES = 2
NUM_SUBCORES = 16
SW_ROWS = N + G


def _tree_max(vals):
    while len(vals) > 1:
        nxt = [jnp.maximum(vals[i], vals[i + 1]) for i in range(0, len(vals) - 1, 2)]
        if len(vals) % 2:
            nxt.append(vals[-1])
        vals = nxt
    return vals[0]


def _cky_body(marg_hbm, lens_hbm, out_hbm, marg_v, sw_v, lens_v, root_v):
    wid = lax.axis_index("s") * NUM_CORES + lax.axis_index("c")

    pltpu.sync_copy(marg_hbm.at[wid], marg_v)
    pltpu.sync_copy(lens_hbm, lens_v)

    iota = lax.iota(jnp.int32, L)
    nmax = jnp.int32(N - 1)
    negv = jnp.full((L,), NEG, jnp.float32)

    def memset_body(t, carry):
        base = t * (16 * L)
        for c in range(16):
            sw_v[pl.ds(base + L * c, L)] = negv
        return carry
    lax.fori_loop(0, SW_ROWS * N // (16 * L), memset_body, jnp.int32(0))

    for c in range(N // L):
        iv = iota + L * c
        jv = jnp.minimum(iv + 1, nmax)
        sw_v[pl.ds((G + 1) * N + L * c, L)] = plsc.load_gather(marg_v, [iv * N + jv])

    def make_band(nb):
        def band_body(w, carry):
            def a_body(t, acar):
                accs, by0, bz0 = acar
                new = list(accs)
                for c in range(nb):
                    terms = []
                    for u in range(U):
                        y = sw_v[pl.ds(by0 + u * N + L * c, L)]
                        z = sw_v[pl.ds(bz0 + u * (1 - N) + L * c, L)]
                        terms.append(y + z)
                    new[c] = jnp.maximum(new[c], _tree_max(terms))
                return (tuple(new), by0 + U * N, bz0 + U * (1 - N))

            accs0 = tuple(jnp.full((L,), NEG, jnp.float32) for _ in range(nb))
            by0 = (G + 1) * N
            bz0 = (G + w - 1) * N + 1
            trips = (w + U - 2) // U
            accs, _, _ = lax.fori_loop(0, trips, a_body, (accs0, by0, bz0))
            for c in range(nb):
                iv = iota + L * c
                jv = jnp.minimum(iv + w, nmax)
                d = plsc.load_gather(marg_v, [iv * N + jv])
                sw_v[pl.ds((G + w) * N + L * c, L)] = accs[c] + d
            return carry

        return band_body

    for nb in range(N // L, 0, -1):
        w_lo = max(2, N - L * nb)
        w_hi = N - L * (nb - 1) - 1
        lax.fori_loop(w_lo, w_hi + 1, make_band(nb), jnp.int32(0))

    widx = jnp.full((L,), wid, jnp.int32)
    lv = plsc.load_gather(lens_v, [widx])
    root_v[...] = plsc.load_gather(sw_v, [(lv + G) * N])
    pltpu.sync_copy(root_v, out_hbm.at[wid])


@jax.jit
def _cky_sc(marginals, lens):
    mesh = plsc.VectorSubcoreMesh(
        core_axis_name="c", subcore_axis_name="s",
        num_cores=NUM_CORES, num_subcores=NUM_SUBCORES,
    )
    k = functools.partial(
        pl.kernel,
        out_type=jax.ShapeDtypeStruct((B, L), jnp.float32),
        mesh=mesh,
        compiler_params=pltpu.CompilerParams(needs_layout_passes=False),
        scratch_types=[
            pltpu.VMEM((N * N,), jnp.float32),
            pltpu.VMEM((SW_ROWS * N,), jnp.float32),
            pltpu.VMEM((B,), jnp.int32),
            pltpu.VMEM((L,), jnp.float32),
        ],
    )(_cky_body)
    return k(marginals.reshape(B, N * N), lens.astype(jnp.int32))


def kernel(marginals, lens):
    out = _cky_sc(marginals, lens)
    return out[:, 0]


# speedup vs baseline: 132.9344x; 1.0773x over previous
import functools

import jax
import jax.numpy as jnp
from jax import lax
from jax.experimental import pallas as pl
from jax.experimental.pallas import tpu as pltpu
from jax.experimental.pallas import tpu_sc as plsc

B, N = 32, 128
L = 16
U = 8
G = U - 2
NEG = -1e9
NUM_CORES = 2
NUM_SUBCORES = 16
SW_ROWS = N + G + 2


def _tree_max(vals):
    while len(vals) > 1:
        nxt = [jnp.maximum(vals[i], vals[i + 1]) for i in range(0, len(vals) - 1, 2)]
        if len(vals) % 2:
            nxt.append(vals[-1])
        vals = nxt
    return vals[0]


def _cky_body(marg_hbm, lens_hbm, out_hbm, marg_v, sw_v, lens_v, root_v):
    wid = lax.axis_index("s") * NUM_CORES + lax.axis_index("c")

    pltpu.sync_copy(marg_hbm.at[wid], marg_v)
    pltpu.sync_copy(lens_hbm, lens_v)

    iota = lax.iota(jnp.int32, L)
    iotaN = iota * N
    nmax = jnp.int32(N - 1)
    negv = jnp.full((L,), NEG, jnp.float32)

    def memset_body(t, carry):
        base = t * (16 * L)
        for c in range(16):
            sw_v[pl.ds(base + L * c, L)] = negv
        return carry
    lax.fori_loop(0, SW_ROWS * N // (16 * L), memset_body, jnp.int32(0))

    for c in range(N // L):
        jv = jnp.minimum(iota + L * c + 1, nmax)
        sw_v[pl.ds((G + 1) * N + L * c, L)] = plsc.load_gather(
            marg_v, [iotaN + L * c * N + jv])

    def make_band(nb, w_lo):
        def pair_body(p, carry):
            w = w_lo + 2 * p

            def a_body(t, acar):
                accA, accB, by0, bz0, bz10 = acar
                newA, newB = list(accA), list(accB)
                for c in range(nb):
                    tA, tB = [], []
                    for u in range(U):
                        y = sw_v[pl.ds(by0 + u * N + L * c, L)]
                        zA = sw_v[pl.ds(bz0 + u * (1 - N) + L * c, L)]
                        zB = sw_v[pl.ds(bz10 + u * (1 - N) + L * c, L)]
                        tA.append(y + zA)
                        tB.append(y + zB)
                    newA[c] = jnp.maximum(newA[c], _tree_max(tA))
                    newB[c] = jnp.maximum(newB[c], _tree_max(tB))
                return (tuple(newA), tuple(newB),
                        by0 + U * N, bz0 + U * (1 - N), bz10 + U * (1 - N))

            accs0 = tuple(jnp.full((L,), NEG, jnp.float32) for _ in range(nb))
            by0 = (G + 1) * N
            bz0 = (G + w - 1) * N + 1
            bz10 = (G + w) * N + 1
            trips = (w + U - 2) // U
            accA, accB, _, _, _ = lax.fori_loop(
                0, trips, a_body, (accs0, accs0, by0, bz0, bz10))

            rowsW = []
            for c in range(nb):
                dA = plsc.load_gather(
                    marg_v, [iotaN + L * c * N
                             + jnp.minimum(iota + L * c + w, nmax)])
                rowW = accA[c] + dA
                sw_v[pl.ds((G + w) * N + L * c, L)] = rowW
                rowsW.append(rowW)
            for c in range(nb):
                z_aw = sw_v[pl.ds((G + 1) * N + L * c + w, L)]
                y_a1 = sw_v[pl.ds((G + 1) * N + L * c, L)]
                z_a1 = sw_v[pl.ds((G + w) * N + L * c + 1, L)]
                xB = jnp.maximum(accB[c],
                                 jnp.maximum(rowsW[c] + z_aw, y_a1 + z_a1))
                dB = plsc.load_gather(
                    marg_v, [iotaN + L * c * N
                             + jnp.minimum(iota + L * c + w + 1, nmax)])
                sw_v[pl.ds((G + w + 1) * N + L * c, L)] = xB + dB
            return carry

        return pair_body

    for nb in range(N // L, 0, -1):
        w_lo = max(2, N - L * nb)
        npairs = (N - L * (nb - 1) - w_lo) // 2
        lax.fori_loop(0, npairs, make_band(nb, w_lo), jnp.int32(0))

    widx = jnp.full((L,), wid, jnp.int32)
    lv = plsc.load_gather(lens_v, [widx])
    root_v[...] = plsc.load_gather(sw_v, [(lv + G) * N])
    pltpu.sync_copy(root_v, out_hbm.at[wid])


@jax.jit
def _cky_sc(marginals, lens):
    mesh = plsc.VectorSubcoreMesh(
        core_axis_name="c", subcore_axis_name="s",
        num_cores=NUM_CORES, num_subcores=NUM_SUBCORES,
    )
    k = functools.partial(
        pl.kernel,
        out_type=jax.ShapeDtypeStruct((B, L), jnp.float32),
        mesh=mesh,
        compiler_params=pltpu.CompilerParams(needs_layout_passes=False),
        scratch_types=[
            pltpu.VMEM((N * N,), jnp.float32),
            pltpu.VMEM((SW_ROWS * N,), jnp.float32),
            pltpu.VMEM((B,), jnp.int32),
            pltpu.VMEM((L,), jnp.float32),
        ],
    )(_cky_body)
    return k(marginals.reshape(B, N * N), lens.astype(jnp.int32))


def kernel(marginals, lens):
    out = _cky_sc(marginals, lens)
    return out[:, 0]


# speedup vs baseline: 140.9025x; 1.0599x over previous
import functools

import jax
import jax.numpy as jnp
from jax import lax
from jax.experimental import pallas as pl
from jax.experimental.pallas import tpu as pltpu
from jax.experimental.pallas import tpu_sc as plsc

B, N = 32, 128
L = 16
U_BY_NB = {1: 16, 2: 16, 3: 8, 4: 8, 5: 8, 6: 8, 7: 4, 8: 2}
U_MAX = 16
G = U_MAX - 2
NEG = -1e9
NUM_CORES = 2
NUM_SUBCORES = 16
SW_ROWS = N + G + 2


def _tree_max(vals):
    while len(vals) > 1:
        nxt = [jnp.maximum(vals[i], vals[i + 1]) for i in range(0, len(vals) - 1, 2)]
        if len(vals) % 2:
            nxt.append(vals[-1])
        vals = nxt
    return vals[0]


def _cky_body(marg_hbm, lens_hbm, out_hbm, marg_v, sw_v, lens_v, root_v):
    wid = lax.axis_index("s") * NUM_CORES + lax.axis_index("c")

    pltpu.sync_copy(marg_hbm.at[wid], marg_v)
    pltpu.sync_copy(lens_hbm, lens_v)

    iota = lax.iota(jnp.int32, L)
    iotaN = iota * N
    nmax = jnp.int32(N - 1)
    negv = jnp.full((L,), NEG, jnp.float32)

    def memset_body(t, carry):
        base = t * (16 * L)
        for c in range(16):
            sw_v[pl.ds(base + L * c, L)] = negv
        return carry
    lax.fori_loop(0, SW_ROWS * N // (16 * L), memset_body, jnp.int32(0))

    for c in range(N // L):
        jv = jnp.minimum(iota + L * c + 1, nmax)
        sw_v[pl.ds((G + 1) * N + L * c, L)] = plsc.load_gather(
            marg_v, [iotaN + L * c * N + jv])

    def make_band(nb, w_lo):
        U = U_BY_NB[nb]

        def pair_body(p, carry):
            w = w_lo + 2 * p

            def a_body(t, acar):
                accA, accB, by0, bz0, bz10 = acar
                newA, newB = list(accA), list(accB)
                for c in range(nb):
                    tA, tB = [], []
                    for u in range(U):
                        y = sw_v[pl.ds(by0 + u * N + L * c, L)]
                        zA = sw_v[pl.ds(bz0 + u * (1 - N) + L * c, L)]
                        zB = sw_v[pl.ds(bz10 + u * (1 - N) + L * c, L)]
                        tA.append(y + zA)
                        tB.append(y + zB)
                    newA[c] = jnp.maximum(newA[c], _tree_max(tA))
                    newB[c] = jnp.maximum(newB[c], _tree_max(tB))
                return (tuple(newA), tuple(newB),
                        by0 + U * N, bz0 + U * (1 - N), bz10 + U * (1 - N))

            accs0 = tuple(jnp.full((L,), NEG, jnp.float32) for _ in range(nb))
            by0 = (G + 1) * N
            bz0 = (G + w - 1) * N + 1
            bz10 = (G + w) * N + 1
            trips = (w + U - 2) // U
            accA, accB, _, _, _ = lax.fori_loop(
                0, trips, a_body, (accs0, accs0, by0, bz0, bz10))

            rowsW = []
            for c in range(nb):
                dA = plsc.load_gather(
                    marg_v, [iotaN + L * c * N
                             + jnp.minimum(iota + L * c + w, nmax)])
                rowW = accA[c] + dA
                sw_v[pl.ds((G + w) * N + L * c, L)] = rowW
                rowsW.append(rowW)
            for c in range(nb):
                z_aw = sw_v[pl.ds((G + 1) * N + L * c + w, L)]
                y_a1 = sw_v[pl.ds((G + 1) * N + L * c, L)]
                z_a1 = sw_v[pl.ds((G + w) * N + L * c + 1, L)]
                xB = jnp.maximum(accB[c],
                                 jnp.maximum(rowsW[c] + z_aw, y_a1 + z_a1))
                dB = plsc.load_gather(
                    marg_v, [iotaN + L * c * N
                             + jnp.minimum(iota + L * c + w + 1, nmax)])
                sw_v[pl.ds((G + w + 1) * N + L * c, L)] = xB + dB
            return carry

        return pair_body

    for nb in range(N // L, 0, -1):
        w_lo = max(2, N - L * nb)
        npairs = (N - L * (nb - 1) - w_lo) // 2
        lax.fori_loop(0, npairs, make_band(nb, w_lo), jnp.int32(0))

    widx = jnp.full((L,), wid, jnp.int32)
    lv = plsc.load_gather(lens_v, [widx])
    root_v[...] = plsc.load_gather(sw_v, [(lv + G) * N])
    pltpu.sync_copy(root_v, out_hbm.at[wid])


@jax.jit
def _cky_sc(marginals, lens):
    mesh = plsc.VectorSubcoreMesh(
        core_axis_name="c", subcore_axis_name="s",
        num_cores=NUM_CORES, num_subcores=NUM_SUBCORES,
    )
    k = functools.partial(
        pl.kernel,
        out_type=jax.ShapeDtypeStruct((B, L), jnp.float32),
        mesh=mesh,
        compiler_params=pltpu.CompilerParams(needs_layout_passes=False),
        scratch_types=[
            pltpu.VMEM((N * N,), jnp.float32),
            pltpu.VMEM((SW_ROWS * N,), jnp.float32),
            pltpu.VMEM((B,), jnp.int32),
            pltpu.VMEM((L,), jnp.float32),
        ],
    )(_cky_body)
    return k(marginals.reshape(B, N * N), lens.astype(jnp.int32))


def kernel(marginals, lens):
    out = _cky_sc(marginals, lens)
    return out[:, 0]
